# Initial kernel scaffold; baseline (speedup 1.0000x reference)
import jax
import jax.numpy as jnp
from jax.experimental import pallas as pl


def kernel(pred_logits, pred_boxes, target_sizes):
    raise NotImplementedError("write your pallas kernel here")


# speedup vs baseline: 8.8571x; 8.8571x over previous
import functools

import jax
import jax.numpy as jnp
from jax import lax
from jax.experimental import pallas as pl
from jax.experimental.pallas import tpu as pltpu
from jax.experimental.pallas import tpu_sc as plsc

K = 300
KPAD = 304
NBINS = 1 << 14
HALF = NBINS // 2
BIN_SHIFT = 18
NCAND = 2048
L = 16


def _sortable(b):
    return b ^ lax.shift_right_logical(b >> 31, 1)


def _make_sc_call(B, Q, C):
    N = Q * C
    NV = N // L
    REM = N - NV * L
    NPAD = (NV + 1) * L
    QB = Q * 4
    inv_c = jnp.float32(1.0 / C)

    mesh = plsc.VectorSubcoreMesh(core_axis_name="c", subcore_axis_name="s")

    @functools.partial(
        pl.kernel,
        out_type=[
            jax.ShapeDtypeStruct((B, KPAD), jnp.float32),
            jax.ShapeDtypeStruct((B, KPAD), jnp.int32),
            jax.ShapeDtypeStruct((B, KPAD * 4), jnp.float32),
        ],
        mesh=mesh,
        compiler_params=pltpu.CompilerParams(needs_layout_passes=False, use_tc_tiling_on_sc=False),
        scratch_types=[
            pltpu.VMEM((NPAD,), jnp.float32),
            pltpu.VMEM((NBINS,), jnp.int32),
            pltpu.VMEM((NCAND + L,), jnp.int32),
            pltpu.VMEM((NCAND + L,), jnp.int32),
            pltpu.VMEM((NCAND,), jnp.int32),
            pltpu.VMEM((QB,), jnp.float32),
            pltpu.VMEM((L,), jnp.float32),
            pltpu.VMEM((KPAD,), jnp.float32),
            pltpu.VMEM((KPAD,), jnp.int32),
            pltpu.VMEM((KPAD * 4,), jnp.float32),
        ],
    )
    def sc_call(logits_hbm, boxes_hbm, scale_hbm,
                out_s_hbm, out_l_hbm, out_b_hbm,
                keys_v, hist_v, cand_s_v, cand_i_v, rank_v,
                boxes_v, scale_v, s_st, l_st, b_st):
        img = lax.axis_index("s") * 2 + lax.axis_index("c")

        pltpu.sync_copy(logits_hbm.at[img], keys_v.at[pl.ds(0, N)])
        pltpu.sync_copy(boxes_hbm.at[img], boxes_v)
        pltpu.sync_copy(scale_hbm.at[img], scale_v)

        lanes = lax.iota(jnp.int32, L)
        ones = jnp.ones((L,), jnp.int32)
        zeros = jnp.zeros((L,), jnp.int32)

        def zero_hist(i, _):
            hist_v[pl.ds(i * L, L)] = zeros
            return 0
        lax.fori_loop(0, NBINS // L, zero_hist, 0)

        def pass1(i, _):
            b = lax.bitcast_convert_type(keys_v[pl.ds(i * L, L)], jnp.int32)
            s = _sortable(b)
            bins = (s >> BIN_SHIFT) + HALF
            plsc.addupdate_scatter(hist_v, [bins], ones)
            return 0
        lax.fori_loop(0, NV, pass1, 0)
        if REM:
            b = lax.bitcast_convert_type(keys_v[pl.ds(NV * L, L)], jnp.int32)
            s = _sortable(b)
            bins = (s >> BIN_SHIFT) + HALF
            m = lanes < REM
            bins = jnp.where(m, bins, 0)
            plsc.addupdate_scatter(hist_v, [bins], ones, mask=m)

        def scan_body(j, carry):
            cum, bstar = carry
            vr = NBINS // L - 1 - j
            v = hist_v[pl.ds(vr * L, L)]
            sfx = jnp.cumsum(lax.rev(v, (0,)))
            tot = jnp.sum(v)
            p = jnp.sum((cum + sfx < K).astype(jnp.int32))
            newcum = cum + tot
            crossed = jnp.logical_and(cum < K, newcum >= K)
            bstar = jnp.where(crossed, vr * L + (L - 1) - p, bstar)
            return newcum, bstar
        _, bstar = lax.fori_loop(0, NBINS // L, scan_body,
                                 (jnp.int32(0), jnp.int32(0)))
        theta = lax.shift_left(bstar - HALF, BIN_SHIFT)

        def compact(i, wptr):
            b = lax.bitcast_convert_type(keys_v[pl.ds(i * L, L)], jnp.int32)
            s = _sortable(b)
            m = s >= theta
            idx = i * L + lanes
            wp = jnp.minimum(wptr, NCAND - L)
            plsc.store_compressed(cand_s_v.at[pl.ds(wp, L)], s, mask=m)
            plsc.store_compressed(cand_i_v.at[pl.ds(wp, L)], idx, mask=m)
            return wptr + jnp.sum(m.astype(jnp.int32))
        n = lax.fori_loop(0, NV, compact, jnp.int32(0))
        if REM:
            b = lax.bitcast_convert_type(keys_v[pl.ds(NV * L, L)], jnp.int32)
            s = _sortable(b)
            m = jnp.logical_and(s >= theta, lanes < REM)
            idx = NV * L + lanes
            wp = jnp.minimum(n, NCAND - L)
            plsc.store_compressed(cand_s_v.at[pl.ds(wp, L)], s, mask=m)
            plsc.store_compressed(cand_i_v.at[pl.ds(wp, L)], idx, mask=m)
            n = n + jnp.sum(m.astype(jnp.int32))

        n = jnp.minimum(n, NCAND)
        cand_s_v[pl.ds(n, L)] = jnp.full((L,), jnp.int32(-(2 ** 31)))
        cand_i_v[pl.ds(n, L)] = jnp.full((L,), jnp.int32(2 ** 31 - 1))
        nv = (n + L - 1) // L

        def rank_outer(iv, _):
            sv = cand_s_v[pl.ds(iv * L, L)]
            ivv = cand_i_v[pl.ds(iv * L, L)]
            def rank_inner(jv, racc):
                sjv = cand_s_v[pl.ds(jv * L, L)]
                ijv = cand_i_v[pl.ds(jv * L, L)]
                for k in range(L):
                    sj = sjv[k]
                    ij = ijv[k]
                    beats = jnp.logical_or(
                        sj > sv, jnp.logical_and(sj == sv, ij < ivv))
                    racc = racc + beats.astype(jnp.int32)
                return racc
            rank_v[pl.ds(iv * L, L)] = lax.fori_loop(0, nv, rank_inner, zeros)
            return 0
        lax.fori_loop(0, nv, rank_outer, 0)

        scale_vec = scale_v[pl.ds(0, L)]
        img_w = scale_vec[0]
        img_h = scale_vec[1]

        def emit(iv, _):
            base = iv * L
            r = rank_v[pl.ds(base, L)]
            s = cand_s_v[pl.ds(base, L)]
            ci = cand_i_v[pl.ds(base, L)]
            m = r < K
            rr = jnp.where(m, r, 0)
            x = lax.bitcast_convert_type(_sortable(s), jnp.float32)
            score = 1.0 / (1.0 + jnp.exp(-x))
            q = ((ci.astype(jnp.float32) + 0.5) * inv_c).astype(jnp.int32)
            q = jnp.where(m, q, 0)
            label = ci - q * C
            qb = q * 4
            cx = plsc.load_gather(boxes_v, [qb])
            cy = plsc.load_gather(boxes_v, [qb + 1])
            w = jnp.maximum(plsc.load_gather(boxes_v, [qb + 2]), 0.0)
            h = jnp.maximum(plsc.load_gather(boxes_v, [qb + 3]), 0.0)
            plsc.store_scatter(s_st, [rr], score, mask=m)
            plsc.store_scatter(l_st, [rr], label, mask=m)
            rb = rr * 4
            plsc.store_scatter(b_st, [rb], (cx - 0.5 * w) * img_w, mask=m)
            plsc.store_scatter(b_st, [rb + 1], (cy - 0.5 * h) * img_h, mask=m)
            plsc.store_scatter(b_st, [rb + 2], (cx + 0.5 * w) * img_w, mask=m)
            plsc.store_scatter(b_st, [rb + 3], (cy + 0.5 * h) * img_h, mask=m)
            return 0
        lax.fori_loop(0, nv, emit, 0)

        pltpu.sync_copy(s_st, out_s_hbm.at[img])
        pltpu.sync_copy(l_st, out_l_hbm.at[img])
        pltpu.sync_copy(b_st, out_b_hbm.at[img])

    return sc_call


def kernel(pred_logits, pred_boxes, target_sizes):
    B, Q, C = pred_logits.shape
    logits2d = pred_logits.reshape(B, Q * C)
    boxes2d = pred_boxes.reshape(B, Q * 4)
    ts = target_sizes.astype(jnp.float32)
    scale2d = jnp.pad(jnp.stack([ts[:, 1], ts[:, 0]], axis=1),
                      ((0, 0), (0, 14)))
    s_pad, l_pad, b_pad = _make_sc_call(B, Q, C)(logits2d, boxes2d, scale2d)
    scores = s_pad[:, :K]
    labels = l_pad[:, :K]
    boxes = b_pad.reshape(B, KPAD, 4)[:, :K, :]
    return scores, labels, boxes


# speedup vs baseline: 11.1074x; 1.2541x over previous
import functools

import jax
import jax.numpy as jnp
from jax import lax
from jax.experimental import pallas as pl
from jax.experimental.pallas import tpu as pltpu
from jax.experimental.pallas import tpu_sc as plsc

K = 300
KPAD = 304
NBINS = 1 << 14
HALF = NBINS // 2
BIN_SHIFT = 18
NCAND = 2048
L = 16
CH = 4
INT_MIN = -(2 ** 31)
INT_MAX = 2 ** 31 - 1
THETA_OPT = 0x40266666


def _sortable(b):
    return b ^ lax.shift_right_logical(b >> 31, 1)


def _make_sc_call(B, Q, C):
    N = Q * C
    NV = N // L
    REM = N - NV * L
    NPAD = (NV + 1) * L
    QB = Q * 4
    inv_c = jnp.float32(1.0 / C)

    mesh = plsc.VectorSubcoreMesh(core_axis_name="c", subcore_axis_name="s")

    @functools.partial(
        pl.kernel,
        out_type=[
            jax.ShapeDtypeStruct((B * KPAD,), jnp.float32),
            jax.ShapeDtypeStruct((B * KPAD,), jnp.int32),
            jax.ShapeDtypeStruct((B * KPAD * 4,), jnp.float32),
        ],
        mesh=mesh,
        compiler_params=pltpu.CompilerParams(
            needs_layout_passes=False, use_tc_tiling_on_sc=False),
        scratch_types=[
            pltpu.VMEM((NPAD,), jnp.float32),
            pltpu.VMEM((NBINS,), jnp.int32),
            pltpu.VMEM((NCAND + L,), jnp.int32),
            pltpu.VMEM((NCAND + L,), jnp.int32),
            pltpu.VMEM((NCAND,), jnp.int32),
            pltpu.VMEM((QB,), jnp.float32),
            pltpu.VMEM((L,), jnp.float32),
            pltpu.VMEM((KPAD,), jnp.float32),
            pltpu.VMEM((KPAD,), jnp.int32),
            pltpu.VMEM((KPAD * 4,), jnp.float32),
        ],
    )
    def sc_call(logits_hbm, boxes_hbm, scale_hbm,
                out_s_hbm, out_l_hbm, out_b_hbm,
                keys_v, hist_v, cand_s_v, cand_i_v, rank_v,
                boxes_v, scale_v, s_st, l_st, b_st):
        img = lax.axis_index("s") * 2 + lax.axis_index("c")

        pltpu.sync_copy(logits_hbm.at[pl.ds(img * N, N)],
                        keys_v.at[pl.ds(0, N)])
        pltpu.sync_copy(boxes_hbm.at[pl.ds(img * QB, QB)], boxes_v)
        pltpu.sync_copy(scale_hbm.at[pl.ds(img * L, L)], scale_v)

        lanes = lax.iota(jnp.int32, L)
        ones = jnp.ones((L,), jnp.int32)
        zeros = jnp.zeros((L,), jnp.int32)

        def zero_hist(i, _):
            hist_v[pl.ds(i * L, L)] = zeros
            return 0
        lax.fori_loop(0, NBINS // L, zero_hist, 0)

        def pass1(i, wptr_v):
            b = lax.bitcast_convert_type(keys_v[pl.ds(i * L, L)], jnp.int32)
            s = _sortable(b)
            bins = (s >> BIN_SHIFT) + HALF
            plsc.addupdate_scatter(hist_v, [bins], ones)
            m = s >= THETA_OPT
            wp = jnp.minimum(wptr_v[0], NCAND - L)
            plsc.store_compressed(cand_s_v.at[pl.ds(wp, L)], s, mask=m)
            plsc.store_compressed(cand_i_v.at[pl.ds(wp, L)],
                                  i * L + lanes, mask=m)
            return wptr_v + plsc.all_reduce_population_count(m)
        wptr_v = lax.fori_loop(0, NV, pass1, zeros)
        if REM:
            b = lax.bitcast_convert_type(keys_v[pl.ds(NV * L, L)], jnp.int32)
            s = _sortable(b)
            valid = lanes < REM
            bins = jnp.where(valid, (s >> BIN_SHIFT) + HALF, 0)
            plsc.addupdate_scatter(hist_v, [bins], ones, mask=valid)
            m = jnp.logical_and(s >= THETA_OPT, valid)
            wp = jnp.minimum(wptr_v[0], NCAND - L)
            plsc.store_compressed(cand_s_v.at[pl.ds(wp, L)], s, mask=m)
            plsc.store_compressed(cand_i_v.at[pl.ds(wp, L)],
                                  NV * L + lanes, mask=m)
            wptr_v = wptr_v + plsc.all_reduce_population_count(m)
        n_opt = wptr_v[0]

        def scan_cond(c):
            prev, cum, vr = c
            return jnp.logical_and(cum < K, vr >= 0)

        def scan_chunk(c):
            prev, cum, vr = c
            tot = jnp.int32(0)
            acc = zeros
            for k in range(CH):
                acc = acc + hist_v[pl.ds((vr + k) * L, L)]
            tot = jnp.sum(acc)
            return cum, cum + tot, vr - CH
        _, _, vr_exit = (0, 0, 0)
        prev, _, vr_exit = lax.while_loop(
            scan_cond, scan_chunk,
            (jnp.int32(0), jnp.int32(0), jnp.int32(NBINS // L - CH)))
        base = vr_exit + CH

        def fine_scan(j, carry):
            cum, bstar = carry
            vr = base + CH - 1 - j
            v = hist_v[pl.ds(vr * L, L)]
            sfx = jnp.cumsum(lax.rev(v, (0,)))
            tot = jnp.sum(v)
            p = jnp.sum((cum + sfx < K).astype(jnp.int32))
            newcum = cum + tot
            crossed = jnp.logical_and(cum < K, newcum >= K)
            bstar = jnp.where(crossed, vr * L + (L - 1) - p, bstar)
            return newcum, bstar
        _, bstar = lax.fori_loop(0, CH, fine_scan, (prev, jnp.int32(0)))
        theta = lax.shift_left(bstar - HALF, BIN_SHIFT)

        good = jnp.logical_and(theta >= THETA_OPT, n_opt <= NCAND - L)

        def mini_compact(_):
            nv_opt = (n_opt + L - 1) // L
            def body(i, wptr_v):
                g = i * L + lanes
                s = cand_s_v[pl.ds(i * L, L)]
                ci = cand_i_v[pl.ds(i * L, L)]
                m = jnp.logical_and(s >= theta, g < n_opt)
                wp = wptr_v[0]
                plsc.store_compressed(cand_s_v.at[pl.ds(wp, L)], s, mask=m)
                plsc.store_compressed(cand_i_v.at[pl.ds(wp, L)], ci, mask=m)
                return wptr_v + plsc.all_reduce_population_count(m)
            return lax.fori_loop(0, nv_opt, body, zeros)[0]

        def full_compact(_):
            def body(i, wptr_v):
                b = lax.bitcast_convert_type(
                    keys_v[pl.ds(i * L, L)], jnp.int32)
                s = _sortable(b)
                m = s >= theta
                wp = jnp.minimum(wptr_v[0], NCAND - L)
                plsc.store_compressed(cand_s_v.at[pl.ds(wp, L)], s, mask=m)
                plsc.store_compressed(cand_i_v.at[pl.ds(wp, L)],
                                      i * L + lanes, mask=m)
                return wptr_v + plsc.all_reduce_population_count(m)
            wv = lax.fori_loop(0, NV, body, zeros)
            if REM:
                b = lax.bitcast_convert_type(
                    keys_v[pl.ds(NV * L, L)], jnp.int32)
                s = _sortable(b)
                m = jnp.logical_and(s >= theta, lanes < REM)
                wp = jnp.minimum(wv[0], NCAND - L)
                plsc.store_compressed(cand_s_v.at[pl.ds(wp, L)], s, mask=m)
                plsc.store_compressed(cand_i_v.at[pl.ds(wp, L)],
                                      NV * L + lanes, mask=m)
                wv = wv + plsc.all_reduce_population_count(m)
            return wv[0]

        n = lax.cond(good, mini_compact, full_compact, 0)
        n = jnp.minimum(n, NCAND)
        cand_s_v[pl.ds(n, L)] = jnp.full((L,), INT_MIN, jnp.int32)
        cand_i_v[pl.ds(n, L)] = jnp.full((L,), INT_MAX, jnp.int32)
        nv = (n + L - 1) // L

        def rank_outer(iv, _):
            sv = cand_s_v[pl.ds(iv * L, L)]
            ivv = cand_i_v[pl.ds(iv * L, L)]
            def rank_inner(jv, racc):
                sjv = cand_s_v[pl.ds(jv * L, L)]
                ijv = cand_i_v[pl.ds(jv * L, L)]
                for k in range(L):
                    sj = sjv[k]
                    ij = ijv[k]
                    beats = jnp.logical_or(
                        sj > sv, jnp.logical_and(sj == sv, ij < ivv))
                    racc = racc + beats.astype(jnp.int32)
                return racc
            rank_v[pl.ds(iv * L, L)] = lax.fori_loop(0, nv, rank_inner, zeros)
            return 0
        lax.fori_loop(0, nv, rank_outer, 0)

        scale_vec = scale_v[pl.ds(0, L)]
        img_w = scale_vec[0]
        img_h = scale_vec[1]

        def emit(iv, _):
            base = iv * L
            r = rank_v[pl.ds(base, L)]
            s = cand_s_v[pl.ds(base, L)]
            ci = cand_i_v[pl.ds(base, L)]
            m = r < K
            rr = jnp.where(m, r, 0)
            x = lax.bitcast_convert_type(_sortable(s), jnp.float32)
            score = 1.0 / (1.0 + jnp.exp(-x))
            q = ((ci.astype(jnp.float32) + 0.5) * inv_c).astype(jnp.int32)
            q = jnp.where(m, q, 0)
            label = ci - q * C
            qb = q * 4
            cx = plsc.load_gather(boxes_v, [qb])
            cy = plsc.load_gather(boxes_v, [qb + 1])
            w = jnp.maximum(plsc.load_gather(boxes_v, [qb + 2]), 0.0)
            h = jnp.maximum(plsc.load_gather(boxes_v, [qb + 3]), 0.0)
            plsc.store_scatter(s_st, [rr], score, mask=m)
            plsc.store_scatter(l_st, [rr], label, mask=m)
            rb = rr * 4
            plsc.store_scatter(b_st, [rb], (cx - 0.5 * w) * img_w, mask=m)
            plsc.store_scatter(b_st, [rb + 1], (cy - 0.5 * h) * img_h, mask=m)
            plsc.store_scatter(b_st, [rb + 2], (cx + 0.5 * w) * img_w, mask=m)
            plsc.store_scatter(b_st, [rb + 3], (cy + 0.5 * h) * img_h, mask=m)
            return 0
        lax.fori_loop(0, nv, emit, 0)

        pltpu.sync_copy(s_st, out_s_hbm.at[pl.ds(img * KPAD, KPAD)])
        pltpu.sync_copy(l_st, out_l_hbm.at[pl.ds(img * KPAD, KPAD)])
        pltpu.sync_copy(b_st, out_b_hbm.at[pl.ds(img * KPAD * 4, KPAD * 4)])

    return sc_call


def kernel(pred_logits, pred_boxes, target_sizes):
    B, Q, C = pred_logits.shape
    logits1d = pred_logits.reshape(B * Q * C)
    boxes1d = pred_boxes.reshape(B * Q * 4)
    ts = target_sizes.astype(jnp.float32)
    scale1d = jnp.pad(jnp.stack([ts[:, 1], ts[:, 0]], axis=1),
                      ((0, 0), (0, L - 2))).reshape(B * L)
    s_pad, l_pad, b_pad = _make_sc_call(B, Q, C)(logits1d, boxes1d, scale1d)
    scores = s_pad.reshape(B, KPAD)[:, :K]
    labels = l_pad.reshape(B, KPAD)[:, :K]
    boxes = b_pad.reshape(B, KPAD, 4)[:, :K, :]
    return scores, labels, boxes


# speedup vs baseline: 11.6833x; 1.0518x over previous
import functools

import jax
import jax.numpy as jnp
from jax import lax
from jax.experimental import pallas as pl
from jax.experimental.pallas import tpu as pltpu
from jax.experimental.pallas import tpu_sc as plsc

K = 300
KPAD = 304
NBINS = 1 << 14
HALF = NBINS // 2
BIN_SHIFT = 18
NCAND = 2048
L = 16
CH = 4
U = 4
INT_MIN = -(2 ** 31)
INT_MAX = 2 ** 31 - 1
THETA_OPT = 0x40266666


def _sortable(b):
    return b ^ lax.shift_right_logical(b >> 31, 1)


def _make_sc_call(B, Q, C):
    N = Q * C
    NV = N // L
    REM = N - NV * L
    NVU = NV // U
    NPAD = (NV + 1) * L
    QB = Q * 4
    inv_c = jnp.float32(1.0 / C)

    mesh = plsc.VectorSubcoreMesh(core_axis_name="c", subcore_axis_name="s")

    @functools.partial(
        pl.kernel,
        out_type=[
            jax.ShapeDtypeStruct((B * KPAD,), jnp.float32),
            jax.ShapeDtypeStruct((B * KPAD,), jnp.int32),
            jax.ShapeDtypeStruct((B * KPAD * 4,), jnp.float32),
        ],
        mesh=mesh,
        compiler_params=pltpu.CompilerParams(
            needs_layout_passes=False, use_tc_tiling_on_sc=False),
        scratch_types=[
            pltpu.VMEM((NPAD,), jnp.int32),
            pltpu.VMEM((NBINS,), jnp.int32),
            pltpu.VMEM((NCAND + L,), jnp.int32),
            pltpu.VMEM((NCAND + L,), jnp.int32),
            pltpu.VMEM((NCAND,), jnp.int32),
            pltpu.VMEM((QB,), jnp.float32),
            pltpu.VMEM((L,), jnp.float32),
            pltpu.VMEM((KPAD,), jnp.float32),
            pltpu.VMEM((KPAD,), jnp.int32),
            pltpu.VMEM((KPAD * 4,), jnp.float32),
        ],
    )
    def sc_call(keys_hbm, boxes_hbm, scale_hbm,
                out_s_hbm, out_l_hbm, out_b_hbm,
                keys_v, hist_v, cand_s_v, cand_i_v, rank_v,
                boxes_v, scale_v, s_st, l_st, b_st):
        img = lax.axis_index("s") * 2 + lax.axis_index("c")

        pltpu.sync_copy(keys_hbm.at[pl.ds(img * N, N)],
                        keys_v.at[pl.ds(0, N)])
        pltpu.sync_copy(boxes_hbm.at[pl.ds(img * QB, QB)], boxes_v)
        pltpu.sync_copy(scale_hbm.at[pl.ds(img * L, L)], scale_v)

        lanes = lax.iota(jnp.int32, L)
        ones = jnp.ones((L,), jnp.int32)
        zeros = jnp.zeros((L,), jnp.int32)

        def zero_hist(i, _):
            for u in range(4):
                hist_v[pl.ds((i * 4 + u) * L, L)] = zeros
            return 0
        lax.fori_loop(0, NBINS // L // 4, zero_hist, 0)

        def p1_step(vi, wptr_v):
            s = keys_v[pl.ds(vi * L, L)]
            bins = (s >> BIN_SHIFT) + HALF
            plsc.addupdate_scatter(hist_v, [bins], ones)
            m = s >= THETA_OPT
            wp = jnp.minimum(wptr_v[0], NCAND - L)
            plsc.store_compressed(cand_i_v.at[pl.ds(wp, L)],
                                  vi * L + lanes, mask=m)
            return wptr_v + plsc.all_reduce_population_count(m)

        def pass1(i, wptr_v):
            for u in range(U):
                wptr_v = p1_step(i * U + u, wptr_v)
            return wptr_v
        wptr_v = lax.fori_loop(0, NVU, pass1, zeros)
        for u in range(NVU * U, NV):
            wptr_v = p1_step(u, wptr_v)
        if REM:
            s = keys_v[pl.ds(NV * L, L)]
            valid = lanes < REM
            bins = jnp.where(valid, (s >> BIN_SHIFT) + HALF, 0)
            plsc.addupdate_scatter(hist_v, [bins], ones, mask=valid)
            m = jnp.logical_and(s >= THETA_OPT, valid)
            wp = jnp.minimum(wptr_v[0], NCAND - L)
            plsc.store_compressed(cand_i_v.at[pl.ds(wp, L)],
                                  NV * L + lanes, mask=m)
            wptr_v = wptr_v + plsc.all_reduce_population_count(m)
        n_opt = wptr_v[0]

        def scan_cond(c):
            prev, cum, vr = c
            return jnp.logical_and(cum < K, vr >= 0)

        def scan_chunk(c):
            prev, cum, vr = c
            acc = zeros
            for k in range(CH):
                acc = acc + hist_v[pl.ds((vr + k) * L, L)]
            return cum, cum + jnp.sum(acc), vr - CH
        prev, _, vr_exit = lax.while_loop(
            scan_cond, scan_chunk,
            (jnp.int32(0), jnp.int32(0), jnp.int32(NBINS // L - CH)))
        cbase = vr_exit + CH

        def fine_scan(j, carry):
            cum, bstar = carry
            vr = cbase + CH - 1 - j
            v = hist_v[pl.ds(vr * L, L)]
            sfx = jnp.cumsum(lax.rev(v, (0,)))
            tot = jnp.sum(v)
            p = jnp.sum((cum + sfx < K).astype(jnp.int32))
            newcum = cum + tot
            crossed = jnp.logical_and(cum < K, newcum >= K)
            bstar = jnp.where(crossed, vr * L + (L - 1) - p, bstar)
            return newcum, bstar
        _, bstar = lax.fori_loop(0, CH, fine_scan, (prev, jnp.int32(0)))
        theta = lax.shift_left(bstar - HALF, BIN_SHIFT)

        good = jnp.logical_and(theta >= THETA_OPT, n_opt <= NCAND - L)

        def mini_compact(_):
            nv_opt = (n_opt + L - 1) // L
            def body(i, wptr_v):
                g = i * L + lanes
                valid = g < n_opt
                ci = cand_i_v[pl.ds(i * L, L)]
                ci = jnp.where(valid, ci, 0)
                s = plsc.load_gather(keys_v, [ci])
                m = jnp.logical_and(s >= theta, valid)
                wp = wptr_v[0]
                plsc.store_compressed(cand_s_v.at[pl.ds(wp, L)], s, mask=m)
                plsc.store_compressed(cand_i_v.at[pl.ds(wp, L)], ci, mask=m)
                return wptr_v + plsc.all_reduce_population_count(m)
            return lax.fori_loop(0, nv_opt, body, zeros)[0]

        def full_compact(_):
            def body(i, wptr_v):
                s = keys_v[pl.ds(i * L, L)]
                m = s >= theta
                wp = jnp.minimum(wptr_v[0], NCAND - L)
                plsc.store_compressed(cand_s_v.at[pl.ds(wp, L)], s, mask=m)
                plsc.store_compressed(cand_i_v.at[pl.ds(wp, L)],
                                      i * L + lanes, mask=m)
                return wptr_v + plsc.all_reduce_population_count(m)
            wv = lax.fori_loop(0, NV, body, zeros)
            if REM:
                s = keys_v[pl.ds(NV * L, L)]
                m = jnp.logical_and(s >= theta, lanes < REM)
                wp = jnp.minimum(wv[0], NCAND - L)
                plsc.store_compressed(cand_s_v.at[pl.ds(wp, L)], s, mask=m)
                plsc.store_compressed(cand_i_v.at[pl.ds(wp, L)],
                                      NV * L + lanes, mask=m)
                wv = wv + plsc.all_reduce_population_count(m)
            return wv[0]

        n = lax.cond(good, mini_compact, full_compact, 0)
        n = jnp.minimum(n, NCAND)
        cand_s_v[pl.ds(n, L)] = jnp.full((L,), INT_MIN, jnp.int32)
        cand_i_v[pl.ds(n, L)] = jnp.full((L,), INT_MAX, jnp.int32)
        nv = (n + L - 1) // L

        def rank_outer(iv, _):
            sv = cand_s_v[pl.ds(iv * L, L)]
            ivv = cand_i_v[pl.ds(iv * L, L)]
            def rank_inner(jv, racc):
                sjv = cand_s_v[pl.ds(jv * L, L)]
                ijv = cand_i_v[pl.ds(jv * L, L)]
                for k in range(L):
                    sj = sjv[k]
                    ij = ijv[k]
                    beats = jnp.logical_or(
                        sj > sv, jnp.logical_and(sj == sv, ij < ivv))
                    racc = racc + beats.astype(jnp.int32)
                return racc
            rank_v[pl.ds(iv * L, L)] = lax.fori_loop(0, nv, rank_inner, zeros)
            return 0
        lax.fori_loop(0, nv, rank_outer, 0)

        scale_vec = scale_v[pl.ds(0, L)]
        img_w = scale_vec[0]
        img_h = scale_vec[1]

        def emit(iv, _):
            base = iv * L
            r = rank_v[pl.ds(base, L)]
            s = cand_s_v[pl.ds(base, L)]
            ci = cand_i_v[pl.ds(base, L)]
            m = r < K
            rr = jnp.where(m, r, 0)
            x = lax.bitcast_convert_type(_sortable(s), jnp.float32)
            score = 1.0 / (1.0 + jnp.exp(-x))
            q = ((ci.astype(jnp.float32) + 0.5) * inv_c).astype(jnp.int32)
            q = jnp.where(m, q, 0)
            label = ci - q * C
            qb = q * 4
            cx = plsc.load_gather(boxes_v, [qb])
            cy = plsc.load_gather(boxes_v, [qb + 1])
            w = jnp.maximum(plsc.load_gather(boxes_v, [qb + 2]), 0.0)
            h = jnp.maximum(plsc.load_gather(boxes_v, [qb + 3]), 0.0)
            plsc.store_scatter(s_st, [rr], score, mask=m)
            plsc.store_scatter(l_st, [rr], label, mask=m)
            rb = rr * 4
            plsc.store_scatter(b_st, [rb], (cx - 0.5 * w) * img_w, mask=m)
            plsc.store_scatter(b_st, [rb + 1], (cy - 0.5 * h) * img_h, mask=m)
            plsc.store_scatter(b_st, [rb + 2], (cx + 0.5 * w) * img_w, mask=m)
            plsc.store_scatter(b_st, [rb + 3], (cy + 0.5 * h) * img_h, mask=m)
            return 0
        lax.fori_loop(0, nv, emit, 0)

        pltpu.sync_copy(s_st, out_s_hbm.at[pl.ds(img * KPAD, KPAD)])
        pltpu.sync_copy(l_st, out_l_hbm.at[pl.ds(img * KPAD, KPAD)])
        pltpu.sync_copy(b_st, out_b_hbm.at[pl.ds(img * KPAD * 4, KPAD * 4)])

    return sc_call


def kernel(pred_logits, pred_boxes, target_sizes):
    B, Q, C = pred_logits.shape
    keys1d = _sortable(
        lax.bitcast_convert_type(pred_logits, jnp.int32)).reshape(B * Q * C)
    boxes1d = pred_boxes.reshape(B * Q * 4)
    ts = target_sizes.astype(jnp.float32)
    scale1d = jnp.pad(jnp.stack([ts[:, 1], ts[:, 0]], axis=1),
                      ((0, 0), (0, L - 2))).reshape(B * L)
    s_pad, l_pad, b_pad = _make_sc_call(B, Q, C)(keys1d, boxes1d, scale1d)
    scores = s_pad.reshape(B, KPAD)[:, :K]
    labels = l_pad.reshape(B, KPAD)[:, :K]
    boxes = b_pad.reshape(B, KPAD, 4)[:, :K, :]
    return scores, labels, boxes
